# Initial kernel scaffold; baseline (speedup 1.0000x reference)
import jax
import jax.numpy as jnp
from jax.experimental import pallas as pl


def kernel(x, edge_index, W1, b1, Wmu, bmu, Wls, bls):
    raise NotImplementedError("write your pallas kernel here")


# speedup vs baseline: 12.2628x; 12.2628x over previous
import functools

import jax
import jax.numpy as jnp
from jax import lax
from jax.experimental import pallas as pl
from jax.experimental.pallas import tpu as pltpu
from jax.experimental.pallas import tpu_sc as plsc

N = 10000
E = 320000
D = 128
DL = 64

NC = 2
NS = 16
NW = NC * NS

CH = 128
EPW = E // NW
NCH = 80
EPW_PAD = NCH * CH
ACC_N = 10112
ROWS_PER_TILE = ACC_N // NS
HCH = NCH // 2

_mesh = lambda: plsc.VectorSubcoreMesh(
    core_axis_name="c", subcore_axis_name="s", num_cores=NC, num_subcores=NS)


def _wid():
    return lax.axis_index("s") * NC + lax.axis_index("c")


def _deg_body(dstb, ones_hbm, zeros_hbm, out, dst_v, ones_v, acc_sh):
    c = lax.axis_index("c")
    s = lax.axis_index("s")
    w = _wid()
    base = s * ROWS_PER_TILE
    pltpu.sync_copy(zeros_hbm, acc_sh.at[pl.ds(base, ROWS_PER_TILE)])
    pltpu.sync_copy(ones_hbm, ones_v)
    pltpu.sync_copy(dstb.at[w], dst_v)
    plsc.subcore_barrier()

    def body(j, carry):
        pltpu.sync_copy(ones_v, acc_sh.at[dst_v.at[j]], add=True)
        return carry

    lax.fori_loop(0, NCH, body, 0)
    plsc.subcore_barrier()
    pltpu.sync_copy(acc_sh.at[pl.ds(base, ROWS_PER_TILE)],
                    out.at[c, pl.ds(base, ROWS_PER_TILE)])


def _deg_call(dstb, onesCH, zerosD):
    return pl.kernel(
        _deg_body,
        out_type=jax.ShapeDtypeStruct((NC, ACC_N, D), jnp.float32),
        mesh=_mesh(),
        scratch_types=[
            pltpu.VMEM((NCH, CH), jnp.int32),
            pltpu.VMEM((CH, D), jnp.float32),
            pltpu.VMEM_SHARED((ACC_N, D), jnp.float32),
        ],
    )(dstb, onesCH, zerosD)


def _make_spmm():
    def body(rows_hbm, srcb, dstb, zeros_hbm, out, src_v, dst_v, buf, acc_sh,
             sem0, sem1):
        c = lax.axis_index("c")
        s = lax.axis_index("s")
        w = _wid()
        pltpu.sync_copy(
            zeros_hbm, acc_sh.at[pl.ds(s * ROWS_PER_TILE, ROWS_PER_TILE)])
        plsc.subcore_barrier()

        def start(j, slot, sem):
            idx = src_v.at[pl.ds(j * CH, CH)]
            pltpu.make_async_copy(rows_hbm.at[idx], buf.at[slot], sem).start()

        def wait(slot, sem):
            pltpu.make_async_copy(rows_hbm.at[src_v.at[pl.ds(0, CH)]],
                                  buf.at[slot], sem).wait()

        def body_fn(i, carry):
            a = 2 * i
            wait(0, sem0)
            start(a + 1, 1, sem1)
            pltpu.sync_copy(buf.at[0], acc_sh.at[dst_v.at[a]], add=True)
            wait(1, sem1)

            @pl.when(a + 2 < HCH)
            def _():
                start(a + 2, 0, sem0)

            pltpu.sync_copy(buf.at[1], acc_sh.at[dst_v.at[a + 1]], add=True)
            return carry

        for half in range(2):
            pltpu.sync_copy(srcb.at[w, pl.ds(half * HCH * CH, HCH * CH)],
                            src_v)
            pltpu.sync_copy(dstb.at[w, pl.ds(half * HCH, HCH)], dst_v)
            start(0, 0, sem0)
            lax.fori_loop(0, HCH // 2, body_fn, 0)
        plsc.subcore_barrier()
        pltpu.sync_copy(acc_sh.at[pl.ds(s * ROWS_PER_TILE, ROWS_PER_TILE)],
                        out.at[c, pl.ds(s * ROWS_PER_TILE, ROWS_PER_TILE)])

    def call(rows, srcb, dstb, zeros640):
        return pl.kernel(
            body,
            out_type=jax.ShapeDtypeStruct((NC, ACC_N, D), jnp.float32),
            mesh=_mesh(),
            scratch_types=[
                pltpu.VMEM((HCH * CH,), jnp.int32),
                pltpu.VMEM((HCH, CH), jnp.int32),
                pltpu.VMEM((2, CH, D), jnp.float32),
                pltpu.VMEM_SHARED((ACC_N, D), jnp.float32),
                pltpu.SemaphoreType.DMA,
                pltpu.SemaphoreType.DMA,
            ],
        )(rows, srcb, dstb, zeros640)

    return call


_spmm_call = _make_spmm()

BN = 1024
GRID = -(-N // BN)


def _tc2_body(degp_ref, x_ref, w_ref, dinv_ref, h_ref, xs_ref):
    deg = degp_ref[0, :, 0:1] + degp_ref[1, :, 0:1] + 1.0
    dv = lax.rsqrt(deg)
    dinv_ref[...] = dv
    h = jnp.dot(x_ref[...], w_ref[...], preferred_element_type=jnp.float32)
    h_ref[...] = h
    xs_ref[...] = dv * h


def _tc2(degp, x, W1):
    return pl.pallas_call(
        _tc2_body,
        grid=(GRID,),
        in_specs=[
            pl.BlockSpec((NC, BN, D), lambda i: (0, i, 0)),
            pl.BlockSpec((BN, D), lambda i: (i, 0)),
            pl.BlockSpec((D, D), lambda i: (0, 0)),
        ],
        out_specs=[
            pl.BlockSpec((BN, 1), lambda i: (i, 0)),
            pl.BlockSpec((BN, D), lambda i: (i, 0)),
            pl.BlockSpec((BN, D), lambda i: (i, 0)),
        ],
        out_shape=[
            jax.ShapeDtypeStruct((N, 1), jnp.float32),
            jax.ShapeDtypeStruct((N, D), jnp.float32),
            jax.ShapeDtypeStruct((N, D), jnp.float32),
        ],
    )(degp, x, W1)


def _tc4_body(p_ref, dinv_ref, h_ref, wcat_ref, b1_ref, hp_ref, hps_ref):
    dv = dinv_ref[...]
    agg = dv * (p_ref[0] + p_ref[1]) + (dv * dv) * h_ref[...]
    hidden = jnp.maximum(agg + b1_ref[...], 0.0)
    hp = jnp.dot(hidden, wcat_ref[...], preferred_element_type=jnp.float32)
    hp_ref[...] = hp
    hps_ref[...] = dv * hp


def _tc4(P, dinv, h, Wcat, b1):
    return pl.pallas_call(
        _tc4_body,
        grid=(GRID,),
        in_specs=[
            pl.BlockSpec((NC, BN, D), lambda i: (0, i, 0)),
            pl.BlockSpec((BN, 1), lambda i: (i, 0)),
            pl.BlockSpec((BN, D), lambda i: (i, 0)),
            pl.BlockSpec((D, D), lambda i: (0, 0)),
            pl.BlockSpec((1, D), lambda i: (0, 0)),
        ],
        out_specs=[
            pl.BlockSpec((BN, D), lambda i: (i, 0)),
            pl.BlockSpec((BN, D), lambda i: (i, 0)),
        ],
        out_shape=[
            jax.ShapeDtypeStruct((N, D), jnp.float32),
            jax.ShapeDtypeStruct((N, D), jnp.float32),
        ],
    )(P, dinv, h, Wcat, b1)


def _tc6_body(q_ref, dinv_ref, hp_ref, bcat_ref, out_ref):
    dv = dinv_ref[...]
    out_ref[...] = (dv * (q_ref[0] + q_ref[1]) + (dv * dv) * hp_ref[...]
                    + bcat_ref[...])


def _tc6(Q, dinv, hp, bcat):
    return pl.pallas_call(
        _tc6_body,
        grid=(GRID,),
        in_specs=[
            pl.BlockSpec((NC, BN, D), lambda i: (0, i, 0)),
            pl.BlockSpec((BN, 1), lambda i: (i, 0)),
            pl.BlockSpec((BN, D), lambda i: (i, 0)),
            pl.BlockSpec((1, D), lambda i: (0, 0)),
        ],
        out_specs=pl.BlockSpec((BN, D), lambda i: (i, 0)),
        out_shape=jax.ShapeDtypeStruct((N, D), jnp.float32),
    )(Q, dinv, hp, bcat)


def kernel(x, edge_index, W1, b1, Wmu, bmu, Wls, bls):
    src = edge_index[0].reshape(NW, EPW)
    dst = edge_index[1].reshape(NW, EPW)
    srcb = jnp.concatenate(
        [src, jnp.zeros((NW, EPW_PAD - EPW), jnp.int32)], axis=1)
    dstf = jnp.concatenate(
        [dst, jnp.full((NW, EPW_PAD - EPW), N, jnp.int32)], axis=1)
    dstb = dstf.reshape(NW, NCH, CH)

    zerosD = jnp.zeros((ROWS_PER_TILE, D), jnp.float32)
    onesCH = jnp.ones((CH, D), jnp.float32)

    Wcat = jnp.concatenate([Wmu, Wls], axis=1)
    bcat = jnp.concatenate([bmu, bls]).reshape(1, D)

    degp = _deg_call(dstb, onesCH, zerosD)
    dinv, h, xs = _tc2(degp, x, W1)
    P = _spmm_call(xs, srcb, dstb, zerosD)
    hp, hps = _tc4(P, dinv, h, Wcat, b1.reshape(1, D))
    Q = _spmm_call(hps, srcb, dstb, zerosD)
    out = _tc6(Q, dinv, hp, bcat)
    return (out[:, :DL], out[:, DL:])


# speedup vs baseline: 13.0549x; 1.0646x over previous
import functools

import jax
import jax.numpy as jnp
from jax import lax
from jax.experimental import pallas as pl
from jax.experimental.pallas import tpu as pltpu
from jax.experimental.pallas import tpu_sc as plsc

N = 10000
E = 320000
D = 128
DL = 64

NC = 2
NS = 16
NW = NC * NS

EPW = E // NW
EPW_PAD = 10240
DCH = 128
DNCH = EPW_PAD // DCH
CH = 64
NCH = EPW_PAD // CH
HCH = NCH // 2
NBUF = 4
ACC_N = 10112
ROWS_PER_TILE = ACC_N // NS

_mesh = lambda: plsc.VectorSubcoreMesh(
    core_axis_name="c", subcore_axis_name="s", num_cores=NC, num_subcores=NS)


def _wid():
    return lax.axis_index("s") * NC + lax.axis_index("c")


def _deg_body(dstb, ones_hbm, zeros_hbm, out, dst_v, ones_v, acc_sh, sem):
    c = lax.axis_index("c")
    s = lax.axis_index("s")
    w = _wid()
    base = s * ROWS_PER_TILE
    pltpu.sync_copy(zeros_hbm, acc_sh.at[pl.ds(base, ROWS_PER_TILE)])
    pltpu.sync_copy(ones_hbm, ones_v)
    pltpu.sync_copy(dstb.at[w], dst_v)
    plsc.subcore_barrier()

    K = 8

    def body(i, carry):
        for k in range(K):
            pltpu.async_copy(ones_v, acc_sh.at[dst_v.at[i * K + k]], sem,
                             add=True)
        for k in range(K):
            pltpu.make_async_copy(ones_v, acc_sh.at[dst_v.at[0]], sem).wait()
        return carry

    lax.fori_loop(0, DNCH // K, body, 0)
    plsc.subcore_barrier()
    pltpu.sync_copy(acc_sh.at[pl.ds(base, ROWS_PER_TILE)],
                    out.at[c, pl.ds(base, ROWS_PER_TILE)])


def _deg_call(dstb, onesCH, zerosD):
    return pl.kernel(
        _deg_body,
        out_type=jax.ShapeDtypeStruct((NC, ACC_N, D), jnp.float32),
        mesh=_mesh(),
        scratch_types=[
            pltpu.VMEM((DNCH, DCH), jnp.int32),
            pltpu.VMEM((DCH, D), jnp.float32),
            pltpu.VMEM_SHARED((ACC_N, D), jnp.float32),
            pltpu.SemaphoreType.DMA,
        ],
    )(dstb, onesCH, zerosD)


def _make_spmm():
    def body(rows_hbm, srcb, dstb, zeros_hbm, out, src_v, dst_v, buf, acc_sh,
             gsems, ssems):
        c = lax.axis_index("c")
        s = lax.axis_index("s")
        w = _wid()
        pltpu.sync_copy(
            zeros_hbm, acc_sh.at[pl.ds(s * ROWS_PER_TILE, ROWS_PER_TILE)])
        plsc.subcore_barrier()

        def gather_start(j, b):
            idx = src_v.at[pl.ds(j * CH, CH)]
            pltpu.make_async_copy(rows_hbm.at[idx], buf.at[b],
                                  gsems[b]).start()

        def gather_wait(b):
            pltpu.make_async_copy(rows_hbm.at[src_v.at[pl.ds(0, CH)]],
                                  buf.at[b], gsems[b]).wait()

        def scatter_start(j, b):
            pltpu.async_copy(buf.at[b], acc_sh.at[dst_v.at[j]], ssems[b],
                             add=True)

        def scatter_wait(b):
            pltpu.make_async_copy(buf.at[b], acc_sh.at[dst_v.at[0]],
                                  ssems[b]).wait()

        def body_fn(i, carry):
            for b in range(NBUF):
                j = i * NBUF + b
                gather_wait(b)
                scatter_start(j, b)

                @pl.when(j + NBUF < HCH)
                def _():
                    scatter_wait(b)
                    gather_start(j + NBUF, b)

            return carry

        for half in range(2):
            pltpu.sync_copy(srcb.at[w, pl.ds(half * HCH * CH, HCH * CH)],
                            src_v)
            pltpu.sync_copy(dstb.at[w, pl.ds(half * HCH, HCH)], dst_v)
            for b in range(NBUF):
                gather_start(b, b)
            lax.fori_loop(0, HCH // NBUF, body_fn, 0)
            for b in range(NBUF):
                scatter_wait(b)
        plsc.subcore_barrier()
        pltpu.sync_copy(acc_sh.at[pl.ds(s * ROWS_PER_TILE, ROWS_PER_TILE)],
                        out.at[c, pl.ds(s * ROWS_PER_TILE, ROWS_PER_TILE)])

    def call(rows, srcb, dstb, zerosD):
        return pl.kernel(
            body,
            out_type=jax.ShapeDtypeStruct((NC, ACC_N, D), jnp.float32),
            mesh=_mesh(),
            scratch_types=[
                pltpu.VMEM((HCH * CH,), jnp.int32),
                pltpu.VMEM((HCH, CH), jnp.int32),
                pltpu.VMEM((NBUF, CH, D), jnp.float32),
                pltpu.VMEM_SHARED((ACC_N, D), jnp.float32),
                [pltpu.SemaphoreType.DMA] * NBUF,
                [pltpu.SemaphoreType.DMA] * NBUF,
            ],
        )(rows, srcb, dstb, zerosD)

    return call


_spmm_call = _make_spmm()

BN = 1024
GRID = -(-N // BN)


def _tc2_body(degp_ref, x_ref, w_ref, dinv_ref, h_ref, xs_ref):
    deg = degp_ref[0, :, 0:1] + degp_ref[1, :, 0:1] + 1.0
    dv = lax.rsqrt(deg)
    dinv_ref[...] = dv
    h = jnp.dot(x_ref[...], w_ref[...], preferred_element_type=jnp.float32)
    h_ref[...] = h
    xs_ref[...] = dv * h


def _tc2(degp, x, W1):
    return pl.pallas_call(
        _tc2_body,
        grid=(GRID,),
        in_specs=[
            pl.BlockSpec((NC, BN, D), lambda i: (0, i, 0)),
            pl.BlockSpec((BN, D), lambda i: (i, 0)),
            pl.BlockSpec((D, D), lambda i: (0, 0)),
        ],
        out_specs=[
            pl.BlockSpec((BN, 1), lambda i: (i, 0)),
            pl.BlockSpec((BN, D), lambda i: (i, 0)),
            pl.BlockSpec((BN, D), lambda i: (i, 0)),
        ],
        out_shape=[
            jax.ShapeDtypeStruct((N, 1), jnp.float32),
            jax.ShapeDtypeStruct((N, D), jnp.float32),
            jax.ShapeDtypeStruct((N, D), jnp.float32),
        ],
    )(degp, x, W1)


def _tc4_body(p_ref, dinv_ref, h_ref, wcat_ref, b1_ref, hp_ref, hps_ref):
    dv = dinv_ref[...]
    agg = dv * (p_ref[0] + p_ref[1]) + (dv * dv) * h_ref[...]
    hidden = jnp.maximum(agg + b1_ref[...], 0.0)
    hp = jnp.dot(hidden, wcat_ref[...], preferred_element_type=jnp.float32)
    hp_ref[...] = hp
    hps_ref[...] = dv * hp


def _tc4(P, dinv, h, Wcat, b1):
    return pl.pallas_call(
        _tc4_body,
        grid=(GRID,),
        in_specs=[
            pl.BlockSpec((NC, BN, D), lambda i: (0, i, 0)),
            pl.BlockSpec((BN, 1), lambda i: (i, 0)),
            pl.BlockSpec((BN, D), lambda i: (i, 0)),
            pl.BlockSpec((D, D), lambda i: (0, 0)),
            pl.BlockSpec((1, D), lambda i: (0, 0)),
        ],
        out_specs=[
            pl.BlockSpec((BN, D), lambda i: (i, 0)),
            pl.BlockSpec((BN, D), lambda i: (i, 0)),
        ],
        out_shape=[
            jax.ShapeDtypeStruct((N, D), jnp.float32),
            jax.ShapeDtypeStruct((N, D), jnp.float32),
        ],
    )(P, dinv, h, Wcat, b1)


def _tc6_body(q_ref, dinv_ref, hp_ref, bcat_ref, out_ref):
    dv = dinv_ref[...]
    out_ref[...] = (dv * (q_ref[0] + q_ref[1]) + (dv * dv) * hp_ref[...]
                    + bcat_ref[...])


def _tc6(Q, dinv, hp, bcat):
    return pl.pallas_call(
        _tc6_body,
        grid=(GRID,),
        in_specs=[
            pl.BlockSpec((NC, BN, D), lambda i: (0, i, 0)),
            pl.BlockSpec((BN, 1), lambda i: (i, 0)),
            pl.BlockSpec((BN, D), lambda i: (i, 0)),
            pl.BlockSpec((1, D), lambda i: (0, 0)),
        ],
        out_specs=pl.BlockSpec((BN, D), lambda i: (i, 0)),
        out_shape=jax.ShapeDtypeStruct((N, D), jnp.float32),
    )(Q, dinv, hp, bcat)


def kernel(x, edge_index, W1, b1, Wmu, bmu, Wls, bls):
    src = edge_index[0].reshape(NW, EPW)
    dst = edge_index[1].reshape(NW, EPW)
    srcb = jnp.concatenate(
        [src, jnp.zeros((NW, EPW_PAD - EPW), jnp.int32)], axis=1)
    dstf = jnp.concatenate(
        [dst, jnp.full((NW, EPW_PAD - EPW), N, jnp.int32)], axis=1)
    dstb_deg = dstf.reshape(NW, DNCH, DCH)
    dstb = dstf.reshape(NW, NCH, CH)

    zerosD = jnp.zeros((ROWS_PER_TILE, D), jnp.float32)
    onesCH = jnp.ones((DCH, D), jnp.float32)

    Wcat = jnp.concatenate([Wmu, Wls], axis=1)
    bcat = jnp.concatenate([bmu, bls]).reshape(1, D)

    degp = _deg_call(dstb_deg, onesCH, zerosD)
    dinv, h, xs = _tc2(degp, x, W1)
    P = _spmm_call(xs, srcb, dstb, zerosD)
    hp, hps = _tc4(P, dinv, h, Wcat, b1.reshape(1, D))
    Q = _spmm_call(hps, srcb, dstb, zerosD)
    out = _tc6(Q, dinv, hp, bcat)
    return (out[:, :DL], out[:, DL:])
